# baseline (speedup 1.0000x reference)
import functools

import jax
import jax.numpy as jnp
from jax import lax
from jax.experimental import pallas as pl
from jax.experimental.pallas import tpu as pltpu
from jax.experimental.pallas import tpu_sc as plsc

_M, _K = 2, 3
_NNZ = 6
_L = 16
_NC, _NS = 2, 16
_NW = _NC * _NS
_N = 1024
_CHUNK = _N // _NW


def _sc_body(y_hbm, rows_hbm, cols_hbm, vals_hbm, out_hbm,
             rows_v, cols_v, vals_v, y_v, o_v, dense_v, sem):
    wid = lax.axis_index("s") * _NC + lax.axis_index("c")
    base = pl.multiple_of(wid * _CHUNK, _CHUNK)

    cps = [
        pltpu.async_copy(rows_hbm, rows_v, sem),
        pltpu.async_copy(cols_hbm, cols_v, sem),
        pltpu.async_copy(vals_hbm, vals_v, sem),
    ]
    for j in range(_K):
        cps.append(pltpu.async_copy(y_hbm.at[j, pl.ds(base, _CHUNK)],
                                    y_v.at[j], sem))
    for cp in cps:
        cp.wait()

    r = rows_v[...]
    c = cols_v[...]
    v = vals_v[...]
    ids = r * _K + c
    dense_v[...] = jnp.zeros((_L,), jnp.float32)
    plsc.addupdate_scatter(dense_v, [ids], v)
    d = dense_v[...]

    def bcast(slot):
        idx = jnp.full((_L, 1), slot, jnp.int32)
        return lax.gather(
            d, idx,
            dimension_numbers=lax.GatherDimensionNumbers(
                offset_dims=(), collapsed_slice_dims=(0,),
                start_index_map=(0,)),
            slice_sizes=(1,),
            mode=lax.GatherScatterMode.PROMISE_IN_BOUNDS)

    coef = [[bcast(i * _K + j) for j in range(_K)] for i in range(_M)]

    yv = [[y_v[j, pl.ds(h * _L, _L)] for h in range(_CHUNK // _L)]
          for j in range(_K)]
    for i in range(_M):
        for h in range(_CHUNK // _L):
            acc = coef[i][0] * yv[0][h]
            for j in range(1, _K):
                acc = acc + coef[i][j] * yv[j][h]
            o_v[i, pl.ds(h * _L, _L)] = acc

    for i in range(_M):
        pltpu.sync_copy(o_v.at[i], out_hbm.at[i, pl.ds(base, _CHUNK)])


def kernel(y, xind, xval):
    xind32 = xind.astype(jnp.int32)
    pad = _L - _NNZ
    rows = jnp.concatenate([xind32[0], jnp.full((pad,), _M, jnp.int32)])
    cols = jnp.concatenate([xind32[1], jnp.full((pad,), _K, jnp.int32)])
    vals = jnp.concatenate([xval, jnp.zeros((pad,), jnp.float32)])

    mesh = plsc.VectorSubcoreMesh(core_axis_name="c", subcore_axis_name="s")
    run = functools.partial(
        pl.kernel,
        mesh=mesh,
        out_type=jax.ShapeDtypeStruct((_M, _N), jnp.float32),
        scratch_types=[
            pltpu.VMEM((_L,), jnp.int32),
            pltpu.VMEM((_L,), jnp.int32),
            pltpu.VMEM((_L,), jnp.float32),
            pltpu.VMEM((_K, _CHUNK), jnp.float32),
            pltpu.VMEM((_M, _CHUNK), jnp.float32),
            pltpu.VMEM((_L,), jnp.float32),
            pltpu.SemaphoreType.DMA,
        ],
        compiler_params=pltpu.CompilerParams(needs_layout_passes=False),
    )(_sc_body)
    return run(y, rows, cols, vals)


# speedup vs baseline: 1.0310x; 1.0310x over previous
import functools

import jax
import jax.numpy as jnp
from jax import lax
from jax.experimental import pallas as pl
from jax.experimental.pallas import tpu as pltpu
from jax.experimental.pallas import tpu_sc as plsc

_M, _K = 2, 3
_NNZ = 6
_L = 16
_NC, _NS = 2, 16
_NW = _NC * _NS
_N = 1024
_CHUNK = _N // _NW


def _sc_body(y_hbm, coo_hbm, out_hbm, coo_v, y_v, o_v, dense_v, sem):
    wid = lax.axis_index("s") * _NC + lax.axis_index("c")
    base = pl.multiple_of(wid * _CHUNK, _CHUNK)

    cps = [pltpu.async_copy(coo_hbm, coo_v, sem)]
    for j in range(_K):
        cps.append(pltpu.async_copy(y_hbm.at[j, pl.ds(base, _CHUNK)],
                                    y_v.at[j], sem))
    for cp in cps:
        cp.wait()

    r = coo_v[0, :]
    c = coo_v[1, :]
    v = lax.bitcast_convert_type(coo_v[2, :], jnp.float32)
    ids = r * _K + c
    dense_v[...] = jnp.zeros((_L,), jnp.float32)
    plsc.addupdate_scatter(dense_v, [ids], v)
    d = dense_v[...]

    def bcast(slot):
        idx = jnp.full((_L, 1), slot, jnp.int32)
        return lax.gather(
            d, idx,
            dimension_numbers=lax.GatherDimensionNumbers(
                offset_dims=(), collapsed_slice_dims=(0,),
                start_index_map=(0,)),
            slice_sizes=(1,),
            mode=lax.GatherScatterMode.PROMISE_IN_BOUNDS)

    coef = [[bcast(i * _K + j) for j in range(_K)] for i in range(_M)]

    yv = [[y_v[j, pl.ds(h * _L, _L)] for h in range(_CHUNK // _L)]
          for j in range(_K)]
    for i in range(_M):
        for h in range(_CHUNK // _L):
            acc = coef[i][0] * yv[0][h]
            for j in range(1, _K):
                acc = acc + coef[i][j] * yv[j][h]
            o_v[i, pl.ds(h * _L, _L)] = acc

    ocps = [pltpu.async_copy(o_v.at[i], out_hbm.at[i, pl.ds(base, _CHUNK)],
                             sem) for i in range(_M)]
    for cp in ocps:
        cp.wait()


def kernel(y, xind, xval):
    xind32 = xind.astype(jnp.int32)
    pad = _L - _NNZ
    coo = jnp.stack([
        jnp.concatenate([xind32[0], jnp.full((pad,), _M, jnp.int32)]),
        jnp.concatenate([xind32[1], jnp.full((pad,), _K, jnp.int32)]),
        jax.lax.bitcast_convert_type(
            jnp.concatenate([xval, jnp.zeros((pad,), jnp.float32)]),
            jnp.int32),
    ])

    mesh = plsc.VectorSubcoreMesh(core_axis_name="c", subcore_axis_name="s")
    run = functools.partial(
        pl.kernel,
        mesh=mesh,
        out_type=jax.ShapeDtypeStruct((_M, _N), jnp.float32),
        scratch_types=[
            pltpu.VMEM((_K, _L), jnp.int32),
            pltpu.VMEM((_K, _CHUNK), jnp.float32),
            pltpu.VMEM((_M, _CHUNK), jnp.float32),
            pltpu.VMEM((_L,), jnp.float32),
            pltpu.SemaphoreType.DMA,
        ],
        compiler_params=pltpu.CompilerParams(needs_layout_passes=False),
    )(_sc_body)
    return run(y, coo)


# speedup vs baseline: 14.1628x; 13.7371x over previous
import jax
import jax.numpy as jnp
from jax.experimental import pallas as pl
from jax.experimental.pallas import tpu as pltpu

_M, _K = 2, 3
_NNZ = 6


def _spmm_kernel(y_ref, xind_ref, xval_ref, out_ref):
    c = [[jnp.float32(0.0)] * _K for _ in range(_M)]
    for k in range(_NNZ):
        r = xind_ref[0, k]
        col = xind_ref[1, k]
        v = xval_ref[k]
        for i in range(_M):
            for j in range(_K):
                hit = jnp.logical_and(r == i, col == j)
                c[i][j] = c[i][j] + jnp.where(hit, v, jnp.float32(0.0))
    yb = y_ref[...]
    for i in range(_M):
        acc = c[i][0] * yb[0:1, :]
        for j in range(1, _K):
            acc = acc + c[i][j] * yb[j : j + 1, :]
        out_ref[i : i + 1, :] = acc


def kernel(y, xind, xval):
    xind32 = xind.astype(jnp.int32)
    return pl.pallas_call(
        _spmm_kernel,
        out_shape=jax.ShapeDtypeStruct((_M, y.shape[1]), y.dtype),
        in_specs=[
            pl.BlockSpec(memory_space=pltpu.VMEM),
            pl.BlockSpec(memory_space=pltpu.SMEM),
            pl.BlockSpec(memory_space=pltpu.SMEM),
        ],
        out_specs=pl.BlockSpec(memory_space=pltpu.VMEM),
    )(y, xind32, xval)
